# baseline (speedup 1.0000x reference)
import jax
import jax.numpy as jnp
from jax import lax
from jax.experimental import pallas as pl
from jax.experimental.pallas import tpu as pltpu
from jax.experimental.pallas import tpu_sc as plsc

B = 1024
SEQ = 200
EMB = 64
NC = 2
NS = 16
NW = NC * NS
ITEMS = B * SEQ
IPT = ITEMS // NW
CHUNK = SEQ
HALF = SEQ // 2
NCHUNK = IPT // CHUNK
NBUF = 4
PRE = 2
LANES = 16


def _fire_chunk(qtab_hbm, ctab_hbm, q_idx_v, c_idx_v, q_buf, o_buf, gsem, j, b):
    for h in range(2):
        dst = o_buf.at[b, pl.ds(h * HALF, HALF)]
        pltpu.async_copy(ctab_hbm.at[c_idx_v.at[2 * j + h]], dst, gsem[b])
        dst = q_buf.at[b, pl.ds(h * HALF, HALF)]
        pltpu.async_copy(qtab_hbm.at[q_idx_v.at[2 * j + h]], dst, gsem[b])


def _wait_chunk(qtab_hbm, ctab_hbm, q_idx_v, c_idx_v, q_buf, o_buf, gsem, j, b):
    for h in range(2):
        dst = o_buf.at[b, pl.ds(h * HALF, HALF)]
        pltpu.make_async_copy(ctab_hbm.at[c_idx_v.at[2 * j + h]], dst, gsem[b]).wait()
        dst = q_buf.at[b, pl.ds(h * HALF, HALF)]
        pltpu.make_async_copy(qtab_hbm.at[q_idx_v.at[2 * j + h]], dst, gsem[b]).wait()


def _body(q_idx_hbm, c_idx_hbm, qtab_hbm, ctab_hbm, pos_hbm, out_hbm,
          q_idx_v, c_idx_v, pos_v, q_buf, o_buf, gsem, osem):
    wid = lax.axis_index("s") * NC + lax.axis_index("c")
    base = wid * IPT
    row0 = wid * (2 * NCHUNK)
    pltpu.sync_copy(q_idx_hbm.at[pl.ds(row0, 2 * NCHUNK)], q_idx_v)
    pltpu.sync_copy(c_idx_hbm.at[pl.ds(row0, 2 * NCHUNK)], c_idx_v)
    pltpu.sync_copy(pos_hbm, pos_v)

    for j in range(PRE):
        _fire_chunk(qtab_hbm, ctab_hbm, q_idx_v, c_idx_v, q_buf, o_buf,
                    gsem, j, j % NBUF)

    @pl.loop(0, NCHUNK, step=NBUF)
    def _turns(k):
        for b in range(NBUF):
            cur = k + b
            _wait_chunk(qtab_hbm, ctab_hbm, q_idx_v, c_idx_v, q_buf, o_buf,
                        gsem, cur, b)

            @pl.loop(0, CHUNK, unroll=8)
            def _item(i):
                for c in range(EMB // LANES):
                    col = c * LANES
                    qv = q_buf[b, i, pl.ds(col, LANES)]
                    pv = pos_v[i, pl.ds(col, LANES)]
                    plsc.addupdate(o_buf.at[b, i, pl.ds(col, LANES)], qv + pv)

            out_dst = out_hbm.at[pl.ds(base + cur * CHUNK, CHUNK)]
            pltpu.async_copy(o_buf.at[b], out_dst, osem[b])

            nxt = cur + PRE
            bn = (b + PRE) % NBUF

            @pl.when(nxt < NCHUNK)
            def _():
                @pl.when(cur >= NBUF - PRE)
                def _():
                    prev = nxt - NBUF
                    src = o_buf.at[bn]
                    dst = out_hbm.at[pl.ds(base + prev * CHUNK, CHUNK)]
                    pltpu.make_async_copy(src, dst, osem[bn]).wait()
                _fire_chunk(qtab_hbm, ctab_hbm, q_idx_v, c_idx_v, q_buf,
                            o_buf, gsem, nxt, bn)

    for j in range(NCHUNK - NBUF, NCHUNK):
        b = j % NBUF
        src = o_buf.at[b]
        dst = out_hbm.at[pl.ds(base + j * CHUNK, CHUNK)]
        pltpu.make_async_copy(src, dst, osem[b]).wait()


def kernel(questions, category, question_table, category_table, pos_table):
    q = questions.reshape(ITEMS // HALF, HALF).astype(jnp.int32)
    c = category.reshape(ITEMS // HALF, HALF).astype(jnp.int32)
    out = pl.kernel(
        _body,
        out_type=jax.ShapeDtypeStruct((ITEMS, EMB), jnp.float32),
        mesh=plsc.VectorSubcoreMesh(core_axis_name="c", subcore_axis_name="s"),
        compiler_params=pltpu.CompilerParams(use_tc_tiling_on_sc=False),
        scratch_types=[
            pltpu.VMEM((2 * NCHUNK, HALF), jnp.int32),
            pltpu.VMEM((2 * NCHUNK, HALF), jnp.int32),
            pltpu.VMEM((SEQ, EMB), jnp.float32),
            pltpu.VMEM((NBUF, CHUNK, EMB), jnp.float32),
            pltpu.VMEM((NBUF, CHUNK, EMB), jnp.float32),
            [pltpu.SemaphoreType.DMA] * NBUF,
            [pltpu.SemaphoreType.DMA] * NBUF,
        ],
    )(q, c, question_table, category_table, pos_table)
    return out.reshape(B, SEQ, EMB)


# speedup vs baseline: 1.0001x; 1.0001x over previous
import jax
import jax.numpy as jnp
from jax import lax
from jax.experimental import pallas as pl
from jax.experimental.pallas import tpu as pltpu
from jax.experimental.pallas import tpu_sc as plsc

B = 1024
SEQ = 200
EMB = 64
NC = 2
NS = 16
NW = NC * NS
ROWS_PER_W = B // NW
HALF = SEQ // 2
NBUF = 4
PRE = 2
LANES = 16


def _fire_chunk(qtab_hbm, ctab_hbm, q_idx_v, c_idx_v, q_buf, o_buf, gsem, j, b):
    for h in range(2):
        dst = o_buf.at[b, pl.ds(h * HALF, HALF)]
        pltpu.async_copy(ctab_hbm.at[c_idx_v.at[2 * j + h]], dst, gsem[b])
        dst = q_buf.at[b, pl.ds(h * HALF, HALF)]
        pltpu.async_copy(qtab_hbm.at[q_idx_v.at[2 * j + h]], dst, gsem[b])


def _wait_chunk(qtab_hbm, ctab_hbm, q_idx_v, c_idx_v, q_buf, o_buf, gsem, j, b):
    for h in range(2):
        dst = o_buf.at[b, pl.ds(h * HALF, HALF)]
        pltpu.make_async_copy(ctab_hbm.at[c_idx_v.at[2 * j + h]], dst,
                              gsem[b]).wait()
        dst = q_buf.at[b, pl.ds(h * HALF, HALF)]
        pltpu.make_async_copy(qtab_hbm.at[q_idx_v.at[2 * j + h]], dst,
                              gsem[b]).wait()


def _body(q_idx_hbm, c_idx_hbm, qtab_hbm, ctab_hbm, pos_hbm, out_hbm,
          q_idx_v, c_idx_v, pos_v, q_buf, o_buf, gsem, osem):
    wid = lax.axis_index("s") * NC + lax.axis_index("c")
    row0 = wid * ROWS_PER_W
    irow0 = wid * (2 * ROWS_PER_W)
    pltpu.sync_copy(q_idx_hbm.at[pl.ds(irow0, 2 * ROWS_PER_W)], q_idx_v)
    pltpu.sync_copy(c_idx_hbm.at[pl.ds(irow0, 2 * ROWS_PER_W)], c_idx_v)
    pltpu.sync_copy(pos_hbm, pos_v)

    for j in range(PRE):
        _fire_chunk(qtab_hbm, ctab_hbm, q_idx_v, c_idx_v, q_buf, o_buf,
                    gsem, j, j % NBUF)

    @pl.loop(0, ROWS_PER_W, step=NBUF)
    def _turns(k):
        for b in range(NBUF):
            cur = k + b
            _wait_chunk(qtab_hbm, ctab_hbm, q_idx_v, c_idx_v, q_buf, o_buf,
                        gsem, cur, b)

            @pl.loop(0, SEQ, unroll=8)
            def _item(i):
                for c in range(EMB // LANES):
                    col = c * LANES
                    qv = q_buf[b, i, pl.ds(col, LANES)]
                    pv = pos_v[i, pl.ds(col, LANES)]
                    plsc.addupdate(o_buf.at[b, i, pl.ds(col, LANES)], qv + pv)

            pltpu.async_copy(o_buf.at[b], out_hbm.at[row0 + cur], osem[b])

            nxt = cur + PRE
            bn = (b + PRE) % NBUF

            @pl.when(nxt < ROWS_PER_W)
            def _():
                @pl.when(cur >= NBUF - PRE)
                def _():
                    prev = nxt - NBUF
                    pltpu.make_async_copy(o_buf.at[bn], out_hbm.at[row0 + prev],
                                          osem[bn]).wait()
                _fire_chunk(qtab_hbm, ctab_hbm, q_idx_v, c_idx_v, q_buf,
                            o_buf, gsem, nxt, bn)

    for j in range(ROWS_PER_W - NBUF, ROWS_PER_W):
        b = j % NBUF
        pltpu.make_async_copy(o_buf.at[b], out_hbm.at[row0 + j], osem[b]).wait()


def kernel(questions, category, question_table, category_table, pos_table):
    out = pl.kernel(
        _body,
        out_type=jax.ShapeDtypeStruct((B, SEQ, EMB), jnp.float32),
        mesh=plsc.VectorSubcoreMesh(core_axis_name="c", subcore_axis_name="s"),
        compiler_params=pltpu.CompilerParams(use_tc_tiling_on_sc=False),
        scratch_types=[
            pltpu.VMEM((2 * ROWS_PER_W, HALF), jnp.int32),
            pltpu.VMEM((2 * ROWS_PER_W, HALF), jnp.int32),
            pltpu.VMEM((SEQ, EMB), jnp.float32),
            pltpu.VMEM((NBUF, SEQ, EMB), jnp.float32),
            pltpu.VMEM((NBUF, SEQ, EMB), jnp.float32),
            [pltpu.SemaphoreType.DMA] * NBUF,
            [pltpu.SemaphoreType.DMA] * NBUF,
        ],
    )(questions.astype(jnp.int32).reshape(B * SEQ // HALF, HALF),
      category.astype(jnp.int32).reshape(B * SEQ // HALF, HALF),
      question_table, category_table, pos_table)
    return out
